# baseline (speedup 1.0000x reference)
import functools

import jax
import jax.numpy as jnp
from jax import lax
from jax.experimental import pallas as pl
from jax.experimental.pallas import tpu as pltpu
from jax.experimental.pallas import tpu_sc as plsc

VOCAB = 1000000
D = 64
DP = 128
B_TOT = 16384 * 50
NC, NS = 2, 16
NW = NC * NS
ROWS_PER_W = B_TOT // NW
SUPER = 256
NSUP = ROWS_PER_W // SUPER
NBUF = 3
IDX_W = 256
IDX_ROWS_PER_W = ROWS_PER_W // IDX_W


def _make_gather():
    mesh = plsc.VectorSubcoreMesh(core_axis_name="c", subcore_axis_name="s")

    @functools.partial(
        pl.kernel,
        mesh=mesh,
        out_type=jax.ShapeDtypeStruct((B_TOT, DP), jnp.float32),
        compiler_params=pltpu.CompilerParams(use_tc_tiling_on_sc=False),
        scratch_types=[
            pltpu.VMEM((IDX_ROWS_PER_W, IDX_W), jnp.int32),
            pltpu.VMEM((SUPER, DP), jnp.float32),
            pltpu.VMEM((SUPER, DP), jnp.float32),
            pltpu.VMEM((SUPER, DP), jnp.float32),
            pltpu.SemaphoreType.DMA,
            pltpu.SemaphoreType.DMA,
            pltpu.SemaphoreType.DMA,
            pltpu.SemaphoreType.DMA,
            pltpu.SemaphoreType.DMA,
            pltpu.SemaphoreType.DMA,
        ],
    )
    def gather_kernel(idx_hbm, table_hbm, out_hbm, idx_v, b0, b1, b2,
                      g0, g1, g2, s0, s1, s2):
        wid = lax.axis_index("s") * NC + lax.axis_index("c")
        pltpu.sync_copy(idx_hbm.at[pl.ds(wid * IDX_ROWS_PER_W,
                                         IDX_ROWS_PER_W)], idx_v)
        base_row = wid * ROWS_PER_W

        bufs = [b0, b1, b2]
        gsems = [g0, g1, g2]
        ssems = [s0, s1, s2]

        def fire(t):
            pltpu.async_copy(table_hbm.at[idx_v.at[t]], bufs[t % NBUF],
                             gsems[t % NBUF])

        def drain(t):
            pltpu.make_async_copy(table_hbm.at[idx_v.at[t]], bufs[t % NBUF],
                                  gsems[t % NBUF]).wait()

        def store(t):
            pltpu.async_copy(bufs[t % NBUF],
                             out_hbm.at[pl.ds(base_row + t * SUPER, SUPER)],
                             ssems[t % NBUF])

        def wait_store(t):
            pltpu.make_async_copy(bufs[t % NBUF],
                                  out_hbm.at[pl.ds(base_row + t * SUPER,
                                                   SUPER)],
                                  ssems[t % NBUF]).wait()

        fire(0)
        fire(1)
        for t in range(NSUP):
            drain(t)
            store(t)
            if t + 2 < NSUP:
                if t >= 1:
                    wait_store(t - 1)
                fire(t + 2)
        for t in range(NSUP - 3, NSUP):
            wait_store(t)

    return gather_kernel


_gather = _make_gather()


@jax.jit
def kernel(x, table):
    idx = x.reshape(B_TOT // IDX_W, IDX_W).astype(jnp.int32)
    table_p = jnp.pad(table, ((0, 0), (0, DP - D)))
    out_p = _gather(idx, table_p)
    return out_p[:, :D].reshape(x.shape[0], x.shape[1], D)


# speedup vs baseline: 1.5390x; 1.5390x over previous
import functools

import jax
import jax.numpy as jnp
from jax import lax
from jax.experimental import pallas as pl
from jax.experimental.pallas import tpu as pltpu
from jax.experimental.pallas import tpu_sc as plsc

VOCAB = 1000000
D = 64
B = 16384
H = 50
HP = 56
DP = 128
NC, NS = 2, 16
NW = NC * NS
BW = B // NW
NBUF = 3
GV = 128
NG = BW // GV


def _make_gather():
    mesh = plsc.VectorSubcoreMesh(core_axis_name="c", subcore_axis_name="s")

    @functools.partial(
        pl.kernel,
        mesh=mesh,
        out_type=jax.ShapeDtypeStruct((B, HP, DP), jnp.float32),
        compiler_params=pltpu.CompilerParams(use_tc_tiling_on_sc=False),
        scratch_types=[
            pltpu.VMEM((H, BW), jnp.int32),
            pltpu.VMEM((BW, D), jnp.float32),
            pltpu.VMEM((BW, D), jnp.float32),
            pltpu.VMEM((BW, D), jnp.float32),
            pltpu.SemaphoreType.DMA,
            pltpu.SemaphoreType.DMA,
            pltpu.SemaphoreType.DMA,
            pltpu.SemaphoreType.DMA,
            pltpu.SemaphoreType.DMA,
            pltpu.SemaphoreType.DMA,
        ],
    )
    def gather_kernel(idx_hbm, table_hbm, out_hbm, idx_v, b0, b1, b2,
                      g0, g1, g2, s0, s1, s2):
        wid = lax.axis_index("s") * NC + lax.axis_index("c")
        base_b = wid * BW
        pltpu.sync_copy(idx_hbm.at[:, pl.ds(base_b, BW)], idx_v)

        bufs = [b0, b1, b2]
        gsems = [g0, g1, g2]
        ssems = [s0, s1, s2]

        def fire(h):
            for j in range(NG):
                pltpu.async_copy(
                    table_hbm.at[idx_v.at[h, pl.ds(j * GV, GV)]],
                    bufs[h % NBUF].at[pl.ds(j * GV, GV)],
                    gsems[h % NBUF])

        def drain(h):
            for j in range(NG):
                pltpu.make_async_copy(
                    table_hbm.at[idx_v.at[h, pl.ds(j * GV, GV)]],
                    bufs[h % NBUF].at[pl.ds(j * GV, GV)],
                    gsems[h % NBUF]).wait()

        def store(h):
            pltpu.async_copy(bufs[h % NBUF],
                             out_hbm.at[pl.ds(base_b, BW), h, pl.ds(0, D)],
                             ssems[h % NBUF])

        def wait_store(h):
            pltpu.make_async_copy(
                bufs[h % NBUF],
                out_hbm.at[pl.ds(base_b, BW), h, pl.ds(0, D)],
                ssems[h % NBUF]).wait()

        fire(0)
        fire(1)
        for h in range(H):
            drain(h)
            store(h)
            if h + 2 < H:
                if h >= 1:
                    wait_store(h - 1)
                fire(h + 2)
        for h in range(H - 3, H):
            wait_store(h)

    return gather_kernel


_gather = _make_gather()


@jax.jit
def kernel(x, table):
    idx_t = x.T.astype(jnp.int32)
    out_p = _gather(idx_t, table)
    return out_p[:, :H, :D]
